# Initial kernel scaffold; baseline (speedup 1.0000x reference)
import jax
import jax.numpy as jnp
from jax.experimental import pallas as pl


def kernel(x):
    raise NotImplementedError("write your pallas kernel here")


# speedup vs baseline: 4.9079x; 4.9079x over previous
import jax
import jax.numpy as jnp
from jax.experimental import pallas as pl

_K = 64
_ROWS_PER_BLOCK = 8


def _topk_mask_body(x_ref, o_ref):
    int_min = jnp.int32(-(2 ** 31))
    xb = x_ref[...]
    b = jax.lax.bitcast_convert_type(xb, jnp.int32)
    key = jnp.where(b < 0, b ^ jnp.int32(0x7FFFFFFF), b)

    def step(i, prefix_u):
        bit = jnp.int32(31) - i
        cand_u = prefix_u | (jnp.int32(1) << bit)
        cand_s = cand_u ^ int_min
        cnt = jnp.sum((key >= cand_s).astype(jnp.int32), axis=1,
                      keepdims=True)
        return jnp.where(cnt >= _K, cand_u, prefix_u)

    prefix_u = jax.lax.fori_loop(
        0, 32, step, jnp.zeros((_ROWS_PER_BLOCK, 1), jnp.int32))
    thresh_s = prefix_u ^ int_min
    o_ref[...] = jnp.where(key >= thresh_s, xb, 0.0)


def kernel(x):
    rows, cols = x.shape
    grid = (rows // _ROWS_PER_BLOCK,)
    return pl.pallas_call(
        _topk_mask_body,
        grid=grid,
        in_specs=[pl.BlockSpec((_ROWS_PER_BLOCK, cols),
                               lambda i: (i, 0))],
        out_specs=pl.BlockSpec((_ROWS_PER_BLOCK, cols), lambda i: (i, 0)),
        out_shape=jax.ShapeDtypeStruct((rows, cols), x.dtype),
    )(x)
